# baseline (speedup 1.0000x reference)
import functools

import numpy as np
import jax
import jax.numpy as jnp
from jax import lax
from jax.experimental import pallas as pl
from jax.experimental.pallas import tpu as pltpu
from jax.experimental.pallas import tpu_sc as plsc

_NUM_FIELDS = 26
_EMBED_DIM = 32
_NUM_NUM = 13
_BATCH = 16384
_FIELD_SIZE = 100000
_OFFSETS = np.arange(_NUM_FIELDS, dtype=np.int32) * _FIELD_SIZE

_NW = 32
_ROWS_PER_W = _BATCH // _NW
_CHUNK = 64
_NCHUNK = _ROWS_PER_W // _CHUNK


def _tc_prep(cat_ref, x_ref, wt_ref, b_ref, off_ref, idx_ref, num_ref):
    idx_ref[...] = cat_ref[...] + off_ref[...]
    num_ref[...] = (
        jnp.dot(x_ref[...], wt_ref[...], preferred_element_type=jnp.float32)
        + b_ref[...]
    )


_sc_mesh = plsc.VectorSubcoreMesh(core_axis_name="c", subcore_axis_name="s")


@functools.partial(
    pl.kernel,
    mesh=_sc_mesh,
    out_type=jax.ShapeDtypeStruct((_BATCH, _NUM_FIELDS + 1, _EMBED_DIM), jnp.float32),
    scratch_types=[
        pltpu.VMEM((_CHUNK, _NUM_FIELDS), jnp.int32),
        pltpu.VMEM((_CHUNK, _NUM_FIELDS + 1, _EMBED_DIM), jnp.float32),
        pltpu.SemaphoreType.DMA,
    ],
    compiler_params=pltpu.CompilerParams(use_tc_tiling_on_sc=False),
)
def _sc_gather(idx_hbm, num_hbm, table_hbm, out_hbm, idx_v, full_v, sem):
    wid = lax.axis_index("s") * 2 + lax.axis_index("c")

    def chunk_body(c, carry):
        base = wid * _ROWS_PER_W + c * _CHUNK
        pltpu.sync_copy(idx_hbm.at[pl.ds(base, _CHUNK)], idx_v)
        cps = [
            pltpu.async_copy(
                table_hbm.at[idx_v.at[r]],
                full_v.at[r, pl.ds(0, _NUM_FIELDS)],
                sem,
            )
            for r in range(_CHUNK)
        ]
        pltpu.sync_copy(
            num_hbm.at[pl.ds(base, _CHUNK)],
            full_v.at[pl.ds(0, _CHUNK), _NUM_FIELDS],
        )
        for cp in cps:
            cp.wait()
        pltpu.sync_copy(full_v, out_hbm.at[pl.ds(base, _CHUNK)])
        return carry

    lax.fori_loop(0, _NCHUNK, chunk_body, 0)


def kernel(categorical_x, numerical_x, table, W, b):
    offsets = jnp.asarray(_OFFSETS)[None, :]
    wt = W.T
    b2 = b[None, :]
    idx, num = pl.pallas_call(
        _tc_prep,
        out_shape=(
            jax.ShapeDtypeStruct((_BATCH, _NUM_FIELDS), jnp.int32),
            jax.ShapeDtypeStruct((_BATCH, _EMBED_DIM), jnp.float32),
        ),
    )(categorical_x, numerical_x, wt, b2, offsets)
    out = _sc_gather(idx, num, table)
    return out.reshape(_BATCH, (_NUM_FIELDS + 1) * _EMBED_DIM)
